# Initial kernel scaffold; baseline (speedup 1.0000x reference)
import jax
import jax.numpy as jnp
from jax.experimental import pallas as pl


def kernel(src_ids, tgt_node_ids, tgt_token_ids, tgt_action_ids, W_src, W_node, W_tok, W_act):
    raise NotImplementedError("write your pallas kernel here")


# speedup vs baseline: 2.7438x; 2.7438x over previous
import functools

import jax
import jax.numpy as jnp
from jax import lax
from jax.experimental import pallas as pl
from jax.experimental.pallas import tpu as pltpu
from jax.experimental.pallas import tpu_sc as plsc

B = 4096
L = 20
D = 64
TOT = B * L
NC = 2
NS = 16
NW = NC * NS
N_PER_W = TOT // NW
CHUNK = 512
NCHUNK = N_PER_W // CHUNK
PAD = 0


def _emb_body(src_hbm, node_hbm, tok_hbm, act_hbm,
              w_src, w_node, w_tok, w_act,
              out_hbm,
              idx_v, rows_v, gsem):
    c = lax.axis_index("c")
    s = lax.axis_index("s")
    wid = s * NC + c
    base = wid * N_PER_W

    pltpu.sync_copy(src_hbm.at[pl.ds(base, N_PER_W)], idx_v.at[0])
    pltpu.sync_copy(node_hbm.at[pl.ds(base, N_PER_W)], idx_v.at[1])
    pltpu.sync_copy(tok_hbm.at[pl.ds(base, N_PER_W)], idx_v.at[2])
    pltpu.sync_copy(act_hbm.at[pl.ds(base, N_PER_W)], idx_v.at[3])

    tables = (w_src, w_node, w_tok, w_act)
    for t in range(4):
        table = tables[t]

        def chunk_body(g, _, t=t, table=table):
            off = g * CHUNK
            pltpu.async_copy(table.at[idx_v.at[t, pl.ds(off, CHUNK)]],
                             rows_v, gsem).wait()

            def grp(j, _):
                iv = idx_v[t, pl.ds(off + j * 16, 16)]
                anyp = jnp.any(iv == PAD)

                @pl.when(anyp)
                def _():
                    m_f = jnp.where(iv == PAD, jnp.float32(0), jnp.float32(1))
                    dnums = lax.GatherDimensionNumbers(
                        offset_dims=(), collapsed_slice_dims=(0,),
                        start_index_map=(0,))
                    for r in range(16):
                        bc = lax.gather(
                            m_f, jnp.full((16, 1), r, jnp.int32),
                            dimension_numbers=dnums, slice_sizes=(1,),
                            mode=lax.GatherScatterMode.PROMISE_IN_BOUNDS)
                        row = j * 16 + r
                        for cc in range(D // 16):
                            sl = pl.ds(cc * 16, 16)
                            rows_v[row, sl] = rows_v[row, sl] * bc
                return 0

            lax.fori_loop(0, CHUNK // 16, grp, 0, unroll=False)

            pltpu.sync_copy(
                rows_v,
                out_hbm.at[pl.ds(base + off, CHUNK), pl.ds(t * D, D)])
            return 0

        lax.fori_loop(0, NCHUNK, chunk_body, 0, unroll=False)


@functools.partial(
    pl.kernel,
    mesh=plsc.VectorSubcoreMesh(core_axis_name="c", subcore_axis_name="s"),
    out_type=jax.ShapeDtypeStruct((TOT, 4 * D), jnp.float32),
    scratch_types=[
        pltpu.VMEM((4, N_PER_W), jnp.int32),
        pltpu.VMEM((CHUNK, D), jnp.float32),
        pltpu.SemaphoreType.DMA,
    ],
    compiler_params=pltpu.CompilerParams(use_tc_tiling_on_sc=False,
                                         needs_layout_passes=False),
)
def _emb_lookup(*refs):
    _emb_body(*refs)


def kernel(src_ids, tgt_node_ids, tgt_token_ids, tgt_action_ids,
           W_src, W_node, W_tok, W_act):
    out = _emb_lookup(src_ids.reshape(-1), tgt_node_ids.reshape(-1),
                      tgt_token_ids.reshape(-1), tgt_action_ids.reshape(-1),
                      W_src, W_node, W_tok, W_act)
    return out.reshape(B, L, 4 * D)


# speedup vs baseline: 2.9171x; 1.0632x over previous
import functools

import jax
import jax.numpy as jnp
from jax import lax
from jax.experimental import pallas as pl
from jax.experimental.pallas import tpu as pltpu
from jax.experimental.pallas import tpu_sc as plsc

B = 4096
L = 20
D = 64
NT = 4
TOT = B * L
NC = 2
NS = 16
NW = NC * NS
N_PER_W = TOT // NW
CHUNK = 160
NCHUNK = N_PER_W // CHUNK
PAD = 0


def _fix_pad_rows(idx_v, rows, t, off, j):
    iv = idx_v[t, pl.ds(off + j * 16, 16)]
    anyp = jnp.any(iv == PAD)

    @pl.when(anyp)
    def _():
        m_f = jnp.where(iv == PAD, jnp.float32(0), jnp.float32(1))
        dnums = lax.GatherDimensionNumbers(
            offset_dims=(), collapsed_slice_dims=(0,), start_index_map=(0,))
        for r in range(16):
            bc = lax.gather(
                m_f, jnp.full((16, 1), r, jnp.int32),
                dimension_numbers=dnums, slice_sizes=(1,),
                mode=lax.GatherScatterMode.PROMISE_IN_BOUNDS)
            row = j * 16 + r
            for cc in range(D // 16):
                sl = pl.ds(cc * 16, 16)
                rows[row, sl] = rows[row, sl] * bc


def _emb_body(src_hbm, node_hbm, tok_hbm, act_hbm,
              w_src, w_node, w_tok, w_act,
              out_hbm,
              idx_v, rows_v, gsem0, gsem1, wsem0, wsem1):
    c = lax.axis_index("c")
    s = lax.axis_index("s")
    wid = s * NC + c
    base = wid * N_PER_W

    pltpu.sync_copy(src_hbm.at[pl.ds(base, N_PER_W)], idx_v.at[0])
    pltpu.sync_copy(node_hbm.at[pl.ds(base, N_PER_W)], idx_v.at[1])
    pltpu.sync_copy(tok_hbm.at[pl.ds(base, N_PER_W)], idx_v.at[2])
    pltpu.sync_copy(act_hbm.at[pl.ds(base, N_PER_W)], idx_v.at[3])

    tables = (w_src, w_node, w_tok, w_act)
    gsems = (gsem0, gsem1)
    wsems = (wsem0, wsem1)

    def start_gathers(i, p):
        off = i * CHUNK
        handles = []
        for t in range(NT):
            handles.append(pltpu.async_copy(
                tables[t].at[idx_v.at[t, pl.ds(off, CHUNK)]],
                rows_v.at[p, t], gsems[p]))
        return handles

    gh = [None, None]
    wh = [None, None]
    gh[0] = start_gathers(0, 0)
    for i in range(NCHUNK):
        p = i % 2
        if i + 1 < NCHUNK:
            if wh[1 - p] is not None:
                for h in wh[1 - p]:
                    h.wait()
                wh[1 - p] = None
            gh[1 - p] = start_gathers(i + 1, 1 - p)
        for h in gh[p]:
            h.wait()
        off = i * CHUNK
        for t in range(NT):
            def grp(j, _, t=t, p=p, off=off):
                _fix_pad_rows(idx_v, rows_v.at[p, t], t, off, j)
                return 0
            lax.fori_loop(0, CHUNK // 16, grp, 0, unroll=False)
        wh[p] = [pltpu.async_copy(
            rows_v.at[p, t],
            out_hbm.at[pl.ds(base + off, CHUNK), pl.ds(t * D, D)],
            wsems[p]) for t in range(NT)]
    for hs in wh:
        if hs is not None:
            for h in hs:
                h.wait()


@functools.partial(
    pl.kernel,
    mesh=plsc.VectorSubcoreMesh(core_axis_name="c", subcore_axis_name="s"),
    out_type=jax.ShapeDtypeStruct((TOT, NT * D), jnp.float32),
    scratch_types=[
        pltpu.VMEM((NT, N_PER_W), jnp.int32),
        pltpu.VMEM((2, NT, CHUNK, D), jnp.float32),
        pltpu.SemaphoreType.DMA,
        pltpu.SemaphoreType.DMA,
        pltpu.SemaphoreType.DMA,
        pltpu.SemaphoreType.DMA,
    ],
    compiler_params=pltpu.CompilerParams(use_tc_tiling_on_sc=False,
                                         needs_layout_passes=False),
)
def _emb_lookup(*refs):
    _emb_body(*refs)


def kernel(src_ids, tgt_node_ids, tgt_token_ids, tgt_action_ids,
           W_src, W_node, W_tok, W_act):
    out = _emb_lookup(src_ids.reshape(-1), tgt_node_ids.reshape(-1),
                      tgt_token_ids.reshape(-1), tgt_action_ids.reshape(-1),
                      W_src, W_node, W_tok, W_act)
    return out.reshape(B, L, NT * D)


# speedup vs baseline: 3.3351x; 1.1433x over previous
import functools

import jax
import jax.numpy as jnp
from jax import lax
from jax.experimental import pallas as pl
from jax.experimental.pallas import tpu as pltpu
from jax.experimental.pallas import tpu_sc as plsc

B = 4096
L = 20
D = 64
NT = 4
NC = 2
NS = 16
NW = NC * NS
BW = B // NW
PAD = 0


def _fix_pad_rows(idx_v, rows, t, l, j):
    iv = idx_v[t, l, pl.ds(j * 16, 16)]
    anyp = jnp.any(iv == PAD)

    @pl.when(anyp)
    def _():
        m_f = jnp.where(iv == PAD, jnp.float32(0), jnp.float32(1))
        dnums = lax.GatherDimensionNumbers(
            offset_dims=(), collapsed_slice_dims=(0,), start_index_map=(0,))
        for r in range(16):
            bc = lax.gather(
                m_f, jnp.full((16, 1), r, jnp.int32),
                dimension_numbers=dnums, slice_sizes=(1,),
                mode=lax.GatherScatterMode.PROMISE_IN_BOUNDS)
            row = j * 16 + r
            for cc in range(D // 16):
                sl = pl.ds(cc * 16, 16)
                rows[row, sl] = rows[row, sl] * bc


def _emb_body(src_hbm, node_hbm, tok_hbm, act_hbm,
              w_src, w_node, w_tok, w_act,
              out_hbm,
              idx_v, rows_v, gsem0, gsem1, gsem2, gsem3,
              wsem0, wsem1, wsem2, wsem3):
    c = lax.axis_index("c")
    s = lax.axis_index("s")
    wid = s * NC + c
    b0 = wid * BW

    ids = (src_hbm, node_hbm, tok_hbm, act_hbm)
    for t in range(NT):
        pltpu.sync_copy(ids[t].at[:, pl.ds(b0, BW)], idx_v.at[t])

    tables = (w_src, w_node, w_tok, w_act)
    gsems = (gsem0, gsem1, gsem2, gsem3)
    wsems = (wsem0, wsem1, wsem2, wsem3)

    def ga(t, l, p):
        return pltpu.make_async_copy(
            tables[t].at[idx_v.at[t, l, :]], rows_v.at[p], gsems[p])

    def wr(t, l, p):
        return pltpu.make_async_copy(
            rows_v.at[p],
            out_hbm.at[l, pl.ds(b0, BW), pl.ds(t * D, D)], wsems[p])

    for t in range(NT):
        ga(t, 0, 0).start()

        def body(k, _, t=t):
            for i in range(4):
                l = 4 * k + i
                p = i
                q = (i + 1) % 4
                if i < 3:
                    @pl.when(k >= 1)
                    def _(t=t, l=l, q=q):
                        wr(t, l - 3, q).wait()
                    ga(t, l + 1, q).start()
                else:
                    wr(t, l - 3, q).wait()

                    @pl.when(k < (L // 4) - 1)
                    def _(t=t, l=l, q=q):
                        ga(t, l + 1, q).start()
                ga(t, l, p).wait()

                def grp(j, _, t=t, l=l, p=p):
                    _fix_pad_rows(idx_v, rows_v.at[p], t, l, j)
                    return 0
                lax.fori_loop(0, BW // 16, grp, 0, unroll=False)
                wr(t, l, p).start()
            return 0

        lax.fori_loop(0, L // 4, body, 0, unroll=False)
        for i in range(1, 4):
            wr(t, L - 4 + i, i).wait()


@functools.partial(
    pl.kernel,
    mesh=plsc.VectorSubcoreMesh(core_axis_name="c", subcore_axis_name="s"),
    out_type=jax.ShapeDtypeStruct((L, B, NT * D), jnp.float32),
    scratch_types=[
        pltpu.VMEM((NT, L, BW), jnp.int32),
        pltpu.VMEM((4, BW, D), jnp.float32),
        pltpu.SemaphoreType.DMA,
        pltpu.SemaphoreType.DMA,
        pltpu.SemaphoreType.DMA,
        pltpu.SemaphoreType.DMA,
        pltpu.SemaphoreType.DMA,
        pltpu.SemaphoreType.DMA,
        pltpu.SemaphoreType.DMA,
        pltpu.SemaphoreType.DMA,
    ],
    compiler_params=pltpu.CompilerParams(use_tc_tiling_on_sc=False,
                                         needs_layout_passes=False),
)
def _emb_lookup(*refs):
    _emb_body(*refs)


def kernel(src_ids, tgt_node_ids, tgt_token_ids, tgt_action_ids,
           W_src, W_node, W_tok, W_act):
    out = _emb_lookup(src_ids.T, tgt_node_ids.T, tgt_token_ids.T,
                      tgt_action_ids.T, W_src, W_node, W_tok, W_act)
    return jnp.transpose(out, (1, 0, 2))
